# baseline (speedup 1.0000x reference)
import jax
import jax.numpy as jnp
from jax import lax
from jax.experimental import pallas as pl
from jax.experimental.pallas import tpu as pltpu

_ALPHA = 0.5
_N_CHUNK = 8
_CHUNK = 1024


def _linear_kernel(x_hbm, w_in_ref, b_in_ref, w_out_ref, b_out_ref, o_hbm,
                   xbuf, obuf, in_sems, out_sems):
    for i in range(_N_CHUNK):
        pltpu.make_async_copy(
            x_hbm.at[pl.ds(i * _CHUNK, _CHUNK), :], xbuf.at[i], in_sems.at[i]
        ).start()
    w = _ALPHA * w_in_ref[...] + (1.0 - _ALPHA) * w_out_ref[...]
    b = _ALPHA * b_in_ref[...] + (1.0 - _ALPHA) * b_out_ref[...]
    for i in range(_N_CHUNK):
        pltpu.make_async_copy(
            x_hbm.at[pl.ds(i * _CHUNK, _CHUNK), :], xbuf.at[i], in_sems.at[i]
        ).wait()
        acc = lax.dot_general(
            xbuf[i], w,
            dimension_numbers=(((1,), (1,)), ((), ())),
            preferred_element_type=jnp.float32,
        )
        obuf[i] = acc + b[None, :]
        pltpu.make_async_copy(
            obuf.at[i], o_hbm.at[pl.ds(i * _CHUNK, _CHUNK), :], out_sems.at[i]
        ).start()
    for i in range(_N_CHUNK):
        pltpu.make_async_copy(
            obuf.at[i], o_hbm.at[pl.ds(i * _CHUNK, _CHUNK), :], out_sems.at[i]
        ).wait()


def kernel(x, At, W_in, b_in, W_out, b_out):
    del At
    Bd, Nd, L = x.shape
    out_ch = W_in.shape[0]
    rows = Bd * Nd
    xf = x.reshape(rows, L)

    out = pl.pallas_call(
        _linear_kernel,
        in_specs=[
            pl.BlockSpec(memory_space=pltpu.MemorySpace.HBM),
            pl.BlockSpec(memory_space=pltpu.MemorySpace.VMEM),
            pl.BlockSpec(memory_space=pltpu.MemorySpace.VMEM),
            pl.BlockSpec(memory_space=pltpu.MemorySpace.VMEM),
            pl.BlockSpec(memory_space=pltpu.MemorySpace.VMEM),
        ],
        out_specs=pl.BlockSpec(memory_space=pltpu.MemorySpace.HBM),
        out_shape=jax.ShapeDtypeStruct((rows, out_ch), jnp.float32),
        scratch_shapes=[
            pltpu.VMEM((_N_CHUNK, _CHUNK, L), jnp.float32),
            pltpu.VMEM((_N_CHUNK, _CHUNK, out_ch), jnp.float32),
            pltpu.SemaphoreType.DMA((_N_CHUNK,)),
            pltpu.SemaphoreType.DMA((_N_CHUNK,)),
        ],
    )(xf, W_in, b_in, W_out, b_out)
    return out.reshape(Bd, Nd, out_ch)


# speedup vs baseline: 1.8217x; 1.8217x over previous
import jax
import jax.numpy as jnp
from jax import lax
from jax.experimental import pallas as pl

_ALPHA = 0.5
_N_BLOCK = 2048


def _linear_kernel(x_ref, w_in_ref, b_in_ref, w_out_ref, b_out_ref, o_ref):
    w = _ALPHA * w_in_ref[...] + (1.0 - _ALPHA) * w_out_ref[...]
    b = _ALPHA * b_in_ref[...] + (1.0 - _ALPHA) * b_out_ref[...]
    acc = lax.dot_general(
        w, x_ref[0],
        dimension_numbers=(((1,), (1,)), ((), ())),
        preferred_element_type=jnp.float32,
    )
    o_ref[0] = acc + b[:, None]


def kernel(x, At, W_in, b_in, W_out, b_out):
    del At
    Bd, Nd, L = x.shape
    out_ch = W_in.shape[0]

    grid = (Bd, Nd // _N_BLOCK)
    out_t = pl.pallas_call(
        _linear_kernel,
        grid=grid,
        in_specs=[
            pl.BlockSpec((1, _N_BLOCK, L), lambda bi, j: (bi, j, 0)),
            pl.BlockSpec((out_ch, L), lambda bi, j: (0, 0)),
            pl.BlockSpec((out_ch,), lambda bi, j: (0,)),
            pl.BlockSpec((out_ch, L), lambda bi, j: (0, 0)),
            pl.BlockSpec((out_ch,), lambda bi, j: (0,)),
        ],
        out_specs=pl.BlockSpec((1, out_ch, _N_BLOCK), lambda bi, j: (bi, 0, j)),
        out_shape=jax.ShapeDtypeStruct((Bd, out_ch, Nd), jnp.float32),
    )(x, W_in, b_in, W_out, b_out)
    return out_t.transpose(0, 2, 1)
